# baseline (speedup 1.0000x reference)
import functools

import jax
import jax.numpy as jnp
from jax.experimental import pallas as pl

N, D, H, E = 10000, 256, 512, 160000
R = 1000
G = N // R
C1 = H // 128
C2 = 2 * C1


def _mm(a, b):
    return jax.lax.dot_general(a, b, (((1,), (0,)), ((), ())),
                               preferred_element_type=jnp.float32)


def _ka_body(x_ref, w1_ref, sout_ref, u_ref, yhat_ref):
    u = _mm(x_ref[...], w1_ref[...])
    u_ref[...] = u
    us = u * sout_ref[...]
    for k in range(C1):
        yhat_ref[k] = us[:, k * 128:(k + 1) * 128]


def _kernel_a(x, w1, s_out):
    return pl.pallas_call(
        _ka_body,
        grid=(G,),
        in_specs=[
            pl.BlockSpec((R, D), lambda i: (i, 0)),
            pl.BlockSpec((D, H), lambda i: (0, 0)),
            pl.BlockSpec((R, 1), lambda i: (i, 0)),
        ],
        out_specs=[
            pl.BlockSpec((R, H), lambda i: (i, 0)),
            pl.BlockSpec((C1, R, 128), lambda i: (0, i, 0)),
        ],
        out_shape=[
            jax.ShapeDtypeStruct((N, H), jnp.float32),
            jax.ShapeDtypeStruct((C1, N, 128), jnp.float32),
        ],
    )(x, w1, s_out)


def _kb_body(t1_ref, u_ref, rin_ref, d_ref, sout_ref, w2_ref,
             glp_ref, ghp_ref, yhat_ref):
    u = u_ref[...]
    s1 = jnp.concatenate([t1_ref[k] for k in range(C1)], axis=1) * rin_ref[...]
    h_lp = jnp.maximum(s1 + d_ref[...] * u, 0.0)
    h_hp = jnp.maximum(u - s1, 0.0)
    w2 = w2_ref[...]
    g_lp = _mm(h_lp, w2)
    g_hp = _mm(h_hp, w2)
    glp_ref[...] = g_lp
    ghp_ref[...] = g_hp
    sout = sout_ref[...]
    gs_lp = g_lp * sout
    gs_hp = g_hp * sout
    for k in range(C1):
        yhat_ref[k] = gs_lp[:, k * 128:(k + 1) * 128]
        yhat_ref[C1 + k] = gs_hp[:, k * 128:(k + 1) * 128]


def _kernel_b(t1, u, r_in, d, s_out, w2):
    return pl.pallas_call(
        _kb_body,
        grid=(G,),
        in_specs=[
            pl.BlockSpec((C1, R, 128), lambda i: (0, i, 0)),
            pl.BlockSpec((R, H), lambda i: (i, 0)),
            pl.BlockSpec((R, 1), lambda i: (i, 0)),
            pl.BlockSpec((R, 1), lambda i: (i, 0)),
            pl.BlockSpec((R, 1), lambda i: (i, 0)),
            pl.BlockSpec((H, H), lambda i: (0, 0)),
        ],
        out_specs=[
            pl.BlockSpec((R, H), lambda i: (i, 0)),
            pl.BlockSpec((R, H), lambda i: (i, 0)),
            pl.BlockSpec((C2, R, 128), lambda i: (0, i, 0)),
        ],
        out_shape=[
            jax.ShapeDtypeStruct((N, H), jnp.float32),
            jax.ShapeDtypeStruct((N, H), jnp.float32),
            jax.ShapeDtypeStruct((C2, N, 128), jnp.float32),
        ],
    )(t1, u, r_in, d, s_out, w2)


def _kc_body(t2_ref, glp_ref, ghp_ref, rin_ref, d_ref,
             p1w_ref, p1b_ref, p2w_ref, p2b_ref,
             zlp_ref, zhp_ref, y_ref, sum_ref, sq_ref):
    rin = rin_ref[...]
    z_lp = (jnp.concatenate([t2_ref[k] for k in range(C1)], axis=1) * rin
            + d_ref[...] * glp_ref[...])
    z_hp = (ghp_ref[...]
            - jnp.concatenate([t2_ref[C1 + k] for k in range(C1)], axis=1) * rin)
    zlp_ref[...] = z_lp
    zhp_ref[...] = z_hp
    p1w, p1b = p1w_ref[...], p1b_ref[...]
    p2w, p2b = p2w_ref[...], p2b_ref[...]
    ys = (_mm(z_lp, p1w) + p1b, _mm(z_hp, p1w) + p1b,
          _mm(z_lp, p2w) + p2b, _mm(z_hp, p2w) + p2b)
    for j in range(4):
        y_ref[j] = ys[j]
        sum_ref[0, j] = jnp.sum(ys[j], axis=0, keepdims=True)
        sq_ref[0, j] = jnp.sum(ys[j] * ys[j], axis=0, keepdims=True)


def _kernel_c(t2, g_lp, g_hp, r_in, d, p1w, p1b, p2w, p2b):
    return pl.pallas_call(
        _kc_body,
        grid=(G,),
        in_specs=[
            pl.BlockSpec((C2, R, 128), lambda i: (0, i, 0)),
            pl.BlockSpec((R, H), lambda i: (i, 0)),
            pl.BlockSpec((R, H), lambda i: (i, 0)),
            pl.BlockSpec((R, 1), lambda i: (i, 0)),
            pl.BlockSpec((R, 1), lambda i: (i, 0)),
            pl.BlockSpec((H, H), lambda i: (0, 0)),
            pl.BlockSpec((1, H), lambda i: (0, 0)),
            pl.BlockSpec((H, H), lambda i: (0, 0)),
            pl.BlockSpec((1, H), lambda i: (0, 0)),
        ],
        out_specs=[
            pl.BlockSpec((R, H), lambda i: (i, 0)),
            pl.BlockSpec((R, H), lambda i: (i, 0)),
            pl.BlockSpec((4, R, H), lambda i: (0, i, 0)),
            pl.BlockSpec((1, 4, 1, H), lambda i: (i, 0, 0, 0)),
            pl.BlockSpec((1, 4, 1, H), lambda i: (i, 0, 0, 0)),
        ],
        out_shape=[
            jax.ShapeDtypeStruct((N, H), jnp.float32),
            jax.ShapeDtypeStruct((N, H), jnp.float32),
            jax.ShapeDtypeStruct((4, N, H), jnp.float32),
            jax.ShapeDtypeStruct((G, 4, 1, H), jnp.float32),
            jax.ShapeDtypeStruct((G, 4, 1, H), jnp.float32),
        ],
    )(t2, g_lp, g_hp, r_in, d, p1w, p1b, p2w, p2b)


def _kd_body(y_ref, sum_ref, sq_ref, gb_ref, a_ref, out_ref):
    inv_n = jnp.float32(1.0 / N)
    for j in range(4):
        mu = jnp.sum(sum_ref[:, j], axis=0, keepdims=False) * inv_n
        var = jnp.sum(sq_ref[:, j], axis=0, keepdims=False) * inv_n - mu * mu
        gamma = gb_ref[2 * j][None]
        beta = gb_ref[2 * j + 1][None]
        a = a_ref[j, 0]
        yn = (y_ref[j] - mu) * jax.lax.rsqrt(var + 1e-5) * gamma + beta
        out_ref[j] = jnp.where(yn > 0, yn, a * yn)


def _kernel_d(y, sums, sqs, gb, avec):
    return pl.pallas_call(
        _kd_body,
        grid=(G,),
        in_specs=[
            pl.BlockSpec((4, R, H), lambda i: (0, i, 0)),
            pl.BlockSpec((G, 4, 1, H), lambda i: (0, 0, 0, 0)),
            pl.BlockSpec((G, 4, 1, H), lambda i: (0, 0, 0, 0)),
            pl.BlockSpec((8, H), lambda i: (0, 0)),
            pl.BlockSpec((4, 1), lambda i: (0, 0)),
        ],
        out_specs=pl.BlockSpec((4, R, H), lambda i: (0, i, 0)),
        out_shape=jax.ShapeDtypeStruct((4, N, H), jnp.float32),
    )(y, sums, sqs, gb, avec)


def kernel(x, edge_index, W1, W2, p1_W, p1_b, p1_gamma, p1_beta, p1_a,
           p2_W, p2_b, p2_gamma, p2_beta, p2_a):
    src = edge_index[0]
    dst = edge_index[1]

    ones_e = jnp.ones((E,), jnp.float32)
    cnt_out = jnp.zeros((N,), jnp.float32).at[src].add(ones_e) + 1.0
    cnt_in = jnp.zeros((N,), jnp.float32).at[dst].add(ones_e) + 1.0

    r_out = jax.lax.rsqrt(cnt_out)[:, None]
    r_in = jax.lax.rsqrt(cnt_in)[:, None]
    d = r_out * r_in

    u, yhat1 = _kernel_a(x, W1, r_out)
    t1 = jnp.zeros((C1, N, 128), jnp.float32).at[:, dst].add(yhat1[:, src])
    g_lp, g_hp, yhat2 = _kernel_b(t1, u, r_in, d, r_out, W2)
    t2 = jnp.zeros((C2, N, 128), jnp.float32).at[:, dst].add(yhat2[:, src])
    z_lp, z_hp, y, sums, sqs = _kernel_c(
        t2, g_lp, g_hp, r_in, d, p1_W, p1_b[None], p2_W, p2_b[None])

    gb = jnp.stack([p1_gamma, p1_beta, p1_gamma, p1_beta,
                    p2_gamma, p2_beta, p2_gamma, p2_beta]).reshape(8, H)
    avec = jnp.stack([p1_a, p1_a, p2_a, p2_a]).reshape(4, 1)
    out = _kernel_d(y, sums, sqs, gb, avec)

    return (out[0], out[1], out[2], out[3], z_lp, z_hp)


# speedup vs baseline: 12.2931x; 12.2931x over previous
import functools

import jax
import jax.numpy as jnp
from jax import lax
from jax.experimental import pallas as pl
from jax.experimental.pallas import tpu as pltpu
from jax.experimental.pallas import tpu_sc as plsc

N, D, H, E = 10000, 256, 512, 160000
R = 1000
G = N // R
C1 = H // 128
C2 = 2 * C1

NSUB = 16
EB = 128
EPAD = 32 * EB * 40
EPT = EPAD // NSUB
NBATCH = EPT // EB
ZR = 632
NPAD = NSUB * ZR
_TAILR = N - (NSUB - 1) * ZR


def _sc_mesh():
    return plsc.VectorSubcoreMesh(core_axis_name="c", subcore_axis_name="s",
                                  num_cores=2, num_subcores=NSUB)


def _make_sc_scatter(K):
    Kh = K // 2

    @functools.partial(
        pl.kernel,
        out_type=jax.ShapeDtypeStruct((K, N, 128), jnp.float32),
        mesh=_sc_mesh(),
        scratch_types=[
            pltpu.VMEM((EB,), jnp.int32),
            pltpu.VMEM((EB,), jnp.int32),
            pltpu.VMEM((EB, 128), jnp.float32),
            pltpu.VMEM_SHARED((NPAD, 128), jnp.float32),
            pltpu.SemaphoreType.DMA,
        ],
    )
    def scatter_k(src_hbm, dst_hbm, zeros_hbm, y_hbm, out_hbm,
                  sidx, didx, rows, acc, sem):
        cid = lax.axis_index("c")
        sid = lax.axis_index("s")
        base = sid * EPT
        row0 = sid * ZR
        for c in range(K):
            @pl.when(cid == (c // Kh))
            def _():
                pltpu.sync_copy(zeros_hbm, acc.at[pl.ds(row0, ZR)])
                plsc.subcore_barrier()

                def body(b, carry):
                    off = base + b * EB
                    pltpu.sync_copy(src_hbm.at[pl.ds(off, EB)], sidx)
                    pltpu.sync_copy(dst_hbm.at[pl.ds(off, EB)], didx)
                    pltpu.async_copy(y_hbm.at[c].at[sidx], rows, sem).wait()
                    pltpu.sync_copy(rows, acc.at[didx], add=True)
                    return carry

                lax.fori_loop(0, NBATCH, body, 0)
                plsc.subcore_barrier()

                @pl.when(sid < NSUB - 1)
                def _():
                    pltpu.sync_copy(acc.at[pl.ds(row0, ZR)],
                                    out_hbm.at[c].at[pl.ds(row0, ZR)])

                @pl.when(sid == NSUB - 1)
                def _():
                    pltpu.sync_copy(
                        acc.at[pl.ds((NSUB - 1) * ZR, _TAILR)],
                        out_hbm.at[c].at[pl.ds((NSUB - 1) * ZR, _TAILR)])

                plsc.subcore_barrier()

    return scatter_k


def _make_sc_count():
    @functools.partial(
        pl.kernel,
        out_type=jax.ShapeDtypeStruct((2, N, 16), jnp.float32),
        mesh=_sc_mesh(),
        scratch_types=[
            pltpu.VMEM((EB,), jnp.int32),
            pltpu.VMEM((EB, 16), jnp.float32),
            pltpu.VMEM_SHARED((NPAD, 16), jnp.float32),
        ],
    )
    def count_k(dstc_hbm, srcc_hbm, ones_hbm, zeros_hbm, out_hbm,
                idxv, onesv, acc):
        cid = lax.axis_index("c")
        sid = lax.axis_index("s")
        pltpu.sync_copy(ones_hbm, onesv)
        base = sid * EPT
        row0 = sid * ZR
        pltpu.sync_copy(zeros_hbm, acc.at[pl.ds(row0, ZR)])
        plsc.subcore_barrier()
        for which in range(2):
            @pl.when(cid == which)
            def _():
                ihbm = dstc_hbm if which == 0 else srcc_hbm

                def body(b, carry):
                    off = base + b * EB
                    pltpu.sync_copy(ihbm.at[pl.ds(off, EB)], idxv)
                    pltpu.sync_copy(onesv, acc.at[idxv], add=True)
                    return carry

                lax.fori_loop(0, NBATCH, body, 0)
                plsc.subcore_barrier()

                @pl.when(sid < NSUB - 1)
                def _():
                    pltpu.sync_copy(acc.at[pl.ds(row0, ZR)],
                                    out_hbm.at[which].at[pl.ds(row0, ZR)])

                @pl.when(sid == NSUB - 1)
                def _():
                    pltpu.sync_copy(
                        acc.at[pl.ds((NSUB - 1) * ZR, _TAILR)],
                        out_hbm.at[which].at[pl.ds((NSUB - 1) * ZR, _TAILR)])

    return count_k


_sc_scatter_cached = functools.lru_cache(maxsize=None)(_make_sc_scatter)
_sc_count_cached = functools.lru_cache(maxsize=None)(_make_sc_count)


def _mm(a, b):
    return jax.lax.dot_general(a, b, (((1,), (0,)), ((), ())),
                               preferred_element_type=jnp.float32)


def _ka_body(x_ref, w1_ref, sout_ref, u_ref, yhat_ref):
    u = _mm(x_ref[...], w1_ref[...])
    u_ref[...] = u
    us = u * sout_ref[...]
    for k in range(C1):
        yhat_ref[k] = us[:, k * 128:(k + 1) * 128]


def _kernel_a(x, w1, s_out):
    return pl.pallas_call(
        _ka_body,
        grid=(G,),
        in_specs=[
            pl.BlockSpec((R, D), lambda i: (i, 0)),
            pl.BlockSpec((D, H), lambda i: (0, 0)),
            pl.BlockSpec((R, 1), lambda i: (i, 0)),
        ],
        out_specs=[
            pl.BlockSpec((R, H), lambda i: (i, 0)),
            pl.BlockSpec((C1, R, 128), lambda i: (0, i, 0)),
        ],
        out_shape=[
            jax.ShapeDtypeStruct((N, H), jnp.float32),
            jax.ShapeDtypeStruct((C1, N, 128), jnp.float32),
        ],
    )(x, w1, s_out)


def _kb_body(t1_ref, u_ref, rin_ref, d_ref, sout_ref, w2_ref,
             glp_ref, ghp_ref, yhat_ref):
    u = u_ref[...]
    s1 = jnp.concatenate([t1_ref[k] for k in range(C1)], axis=1) * rin_ref[...]
    h_lp = jnp.maximum(s1 + d_ref[...] * u, 0.0)
    h_hp = jnp.maximum(u - s1, 0.0)
    w2 = w2_ref[...]
    g_lp = _mm(h_lp, w2)
    g_hp = _mm(h_hp, w2)
    glp_ref[...] = g_lp
    ghp_ref[...] = g_hp
    sout = sout_ref[...]
    gs_lp = g_lp * sout
    gs_hp = g_hp * sout
    for k in range(C1):
        yhat_ref[k] = gs_lp[:, k * 128:(k + 1) * 128]
        yhat_ref[C1 + k] = gs_hp[:, k * 128:(k + 1) * 128]


def _kernel_b(t1, u, r_in, d, s_out, w2):
    return pl.pallas_call(
        _kb_body,
        grid=(G,),
        in_specs=[
            pl.BlockSpec((C1, R, 128), lambda i: (0, i, 0)),
            pl.BlockSpec((R, H), lambda i: (i, 0)),
            pl.BlockSpec((R, 1), lambda i: (i, 0)),
            pl.BlockSpec((R, 1), lambda i: (i, 0)),
            pl.BlockSpec((R, 1), lambda i: (i, 0)),
            pl.BlockSpec((H, H), lambda i: (0, 0)),
        ],
        out_specs=[
            pl.BlockSpec((R, H), lambda i: (i, 0)),
            pl.BlockSpec((R, H), lambda i: (i, 0)),
            pl.BlockSpec((C2, R, 128), lambda i: (0, i, 0)),
        ],
        out_shape=[
            jax.ShapeDtypeStruct((N, H), jnp.float32),
            jax.ShapeDtypeStruct((N, H), jnp.float32),
            jax.ShapeDtypeStruct((C2, N, 128), jnp.float32),
        ],
    )(t1, u, r_in, d, s_out, w2)


def _kc_body(t2_ref, glp_ref, ghp_ref, rin_ref, d_ref,
             p1w_ref, p1b_ref, p2w_ref, p2b_ref,
             zlp_ref, zhp_ref, y_ref, sum_ref, sq_ref):
    rin = rin_ref[...]
    z_lp = (jnp.concatenate([t2_ref[k] for k in range(C1)], axis=1) * rin
            + d_ref[...] * glp_ref[...])
    z_hp = (ghp_ref[...]
            - jnp.concatenate([t2_ref[C1 + k] for k in range(C1)], axis=1) * rin)
    zlp_ref[...] = z_lp
    zhp_ref[...] = z_hp
    p1w, p1b = p1w_ref[...], p1b_ref[...]
    p2w, p2b = p2w_ref[...], p2b_ref[...]
    ys = (_mm(z_lp, p1w) + p1b, _mm(z_hp, p1w) + p1b,
          _mm(z_lp, p2w) + p2b, _mm(z_hp, p2w) + p2b)
    for j in range(4):
        y_ref[j] = ys[j]
        sum_ref[0, j] = jnp.sum(ys[j], axis=0, keepdims=True)
        sq_ref[0, j] = jnp.sum(ys[j] * ys[j], axis=0, keepdims=True)


def _kernel_c(t2, g_lp, g_hp, r_in, d, p1w, p1b, p2w, p2b):
    return pl.pallas_call(
        _kc_body,
        grid=(G,),
        in_specs=[
            pl.BlockSpec((C2, R, 128), lambda i: (0, i, 0)),
            pl.BlockSpec((R, H), lambda i: (i, 0)),
            pl.BlockSpec((R, H), lambda i: (i, 0)),
            pl.BlockSpec((R, 1), lambda i: (i, 0)),
            pl.BlockSpec((R, 1), lambda i: (i, 0)),
            pl.BlockSpec((H, H), lambda i: (0, 0)),
            pl.BlockSpec((1, H), lambda i: (0, 0)),
            pl.BlockSpec((H, H), lambda i: (0, 0)),
            pl.BlockSpec((1, H), lambda i: (0, 0)),
        ],
        out_specs=[
            pl.BlockSpec((R, H), lambda i: (i, 0)),
            pl.BlockSpec((R, H), lambda i: (i, 0)),
            pl.BlockSpec((4, R, H), lambda i: (0, i, 0)),
            pl.BlockSpec((1, 4, 1, H), lambda i: (i, 0, 0, 0)),
            pl.BlockSpec((1, 4, 1, H), lambda i: (i, 0, 0, 0)),
        ],
        out_shape=[
            jax.ShapeDtypeStruct((N, H), jnp.float32),
            jax.ShapeDtypeStruct((N, H), jnp.float32),
            jax.ShapeDtypeStruct((4, N, H), jnp.float32),
            jax.ShapeDtypeStruct((G, 4, 1, H), jnp.float32),
            jax.ShapeDtypeStruct((G, 4, 1, H), jnp.float32),
        ],
    )(t2, g_lp, g_hp, r_in, d, p1w, p1b, p2w, p2b)


def _kd_body(y_ref, sum_ref, sq_ref, gb_ref, a_ref, out_ref):
    inv_n = jnp.float32(1.0 / N)
    for j in range(4):
        mu = jnp.sum(sum_ref[:, j], axis=0, keepdims=False) * inv_n
        var = jnp.sum(sq_ref[:, j], axis=0, keepdims=False) * inv_n - mu * mu
        gamma = gb_ref[2 * j][None]
        beta = gb_ref[2 * j + 1][None]
        a = a_ref[j, 0]
        yn = (y_ref[j] - mu) * jax.lax.rsqrt(var + 1e-5) * gamma + beta
        out_ref[j] = jnp.where(yn > 0, yn, a * yn)


def _kernel_d(y, sums, sqs, gb, avec):
    return pl.pallas_call(
        _kd_body,
        grid=(G,),
        in_specs=[
            pl.BlockSpec((4, R, H), lambda i: (0, i, 0)),
            pl.BlockSpec((G, 4, 1, H), lambda i: (0, 0, 0, 0)),
            pl.BlockSpec((G, 4, 1, H), lambda i: (0, 0, 0, 0)),
            pl.BlockSpec((8, H), lambda i: (0, 0)),
            pl.BlockSpec((4, 1), lambda i: (0, 0)),
        ],
        out_specs=pl.BlockSpec((4, R, H), lambda i: (0, i, 0)),
        out_shape=jax.ShapeDtypeStruct((4, N, H), jnp.float32),
    )(y, sums, sqs, gb, avec)


def kernel(x, edge_index, W1, W2, p1_W, p1_b, p1_gamma, p1_beta, p1_a,
           p2_W, p2_b, p2_gamma, p2_beta, p2_a):
    src = edge_index[0].astype(jnp.int32)
    dst = edge_index[1].astype(jnp.int32)

    npadE = EPAD - E
    src_pad = jnp.concatenate([src, jnp.zeros((npadE,), jnp.int32)])
    dst_pad = jnp.concatenate([dst, jnp.full((npadE,), N, jnp.int32)])
    src_cnt = jnp.concatenate([src, jnp.full((npadE,), N, jnp.int32)])

    ones16 = jnp.ones((EB, 16), jnp.float32)
    zeros16 = jnp.zeros((ZR, 16), jnp.float32)
    zeros128 = jnp.zeros((ZR, 128), jnp.float32)

    cnts = _sc_count_cached()(dst_pad, src_cnt, ones16, zeros16)
    cnt_in = cnts[0, :, 0] + 1.0
    cnt_out = cnts[1, :, 0] + 1.0

    r_out = jax.lax.rsqrt(cnt_out)[:, None]
    r_in = jax.lax.rsqrt(cnt_in)[:, None]
    d = r_out * r_in

    u, yhat1 = _kernel_a(x, W1, r_out)
    t1 = _sc_scatter_cached(C1)(src_pad, dst_pad, zeros128, yhat1)
    g_lp, g_hp, yhat2 = _kernel_b(t1, u, r_in, d, r_out, W2)
    t2 = _sc_scatter_cached(C2)(src_pad, dst_pad, zeros128, yhat2)
    z_lp, z_hp, y, sums, sqs = _kernel_c(
        t2, g_lp, g_hp, r_in, d, p1_W, p1_b[None], p2_W, p2_b[None])

    gb = jnp.stack([p1_gamma, p1_beta, p1_gamma, p1_beta,
                    p2_gamma, p2_beta, p2_gamma, p2_beta]).reshape(8, H)
    avec = jnp.stack([p1_a, p1_a, p2_a, p2_a]).reshape(4, 1)
    out = _kernel_d(y, sums, sqs, gb, avec)

    return (out[0], out[1], out[2], out[3], z_lp, z_hp)


# speedup vs baseline: 14.0295x; 1.1412x over previous
import functools

import jax
import jax.numpy as jnp
from jax import lax
from jax.experimental import pallas as pl
from jax.experimental.pallas import tpu as pltpu
from jax.experimental.pallas import tpu_sc as plsc

N, D, H, E = 10000, 256, 512, 160000
R = 1000
G = N // R
C1 = H // 128
C2 = 2 * C1

NSUB = 16
EB = 128
EPAD = 32 * EB * 40
EPT = EPAD // NSUB
NBATCH = EPT // EB
ZR = 632
NPAD = NSUB * ZR
_TAILR = N - (NSUB - 1) * ZR


def _sc_mesh():
    return plsc.VectorSubcoreMesh(core_axis_name="c", subcore_axis_name="s",
                                  num_cores=2, num_subcores=NSUB)


def _make_sc_scatter(K):
    Kh = K // 2

    @functools.partial(
        pl.kernel,
        out_type=jax.ShapeDtypeStruct((K, N, 128), jnp.float32),
        mesh=_sc_mesh(),
        scratch_types=[
            pltpu.VMEM((EB,), jnp.int32),
            pltpu.VMEM((EB,), jnp.int32),
            pltpu.VMEM((EB,), jnp.int32),
            pltpu.VMEM((EB,), jnp.int32),
            pltpu.VMEM((EB, 128), jnp.float32),
            pltpu.VMEM((EB, 128), jnp.float32),
            pltpu.VMEM_SHARED((NPAD, 128), jnp.float32),
            pltpu.SemaphoreType.DMA,
        ],
    )
    def scatter_k(src_hbm, dst_hbm, zeros_hbm, y_hbm, out_hbm,
                  sidx0, sidx1, didx0, didx1, rows0, rows1, acc, sem):
        cid = lax.axis_index("c")
        sid = lax.axis_index("s")
        row0 = sid * ZR
        base = sid * EPT
        sbuf = (sidx0, sidx1)
        dbuf = (didx0, didx1)
        rbuf = (rows0, rows1)

        for c in range(K):
            @pl.when(cid == (c // Kh))
            def _():
                pltpu.sync_copy(zeros_hbm, acc.at[pl.ds(row0, ZR)])
                plsc.subcore_barrier()
                yc = y_hbm.at[c]

                def body2(i, carry):
                    o0 = base + (2 * i) * EB
                    o1 = o0 + EB
                    i0 = pltpu.async_copy(src_hbm.at[pl.ds(o0, EB)],
                                          sbuf[0], sem)
                    i1 = pltpu.async_copy(dst_hbm.at[pl.ds(o0, EB)],
                                          dbuf[0], sem)
                    i2 = pltpu.async_copy(src_hbm.at[pl.ds(o1, EB)],
                                          sbuf[1], sem)
                    i3 = pltpu.async_copy(dst_hbm.at[pl.ds(o1, EB)],
                                          dbuf[1], sem)
                    i0.wait()
                    i1.wait()
                    i2.wait()
                    i3.wait()
                    g0 = pltpu.async_copy(yc.at[sbuf[0]], rbuf[0], sem)
                    g1 = pltpu.async_copy(yc.at[sbuf[1]], rbuf[1], sem)
                    g0.wait()
                    g1.wait()
                    pltpu.sync_copy(rbuf[0], acc.at[dbuf[0]], add=True)
                    pltpu.sync_copy(rbuf[1], acc.at[dbuf[1]], add=True)
                    return carry

                lax.fori_loop(0, NBATCH // 2, body2, 0)
                plsc.subcore_barrier()

                @pl.when(sid < NSUB - 1)
                def _():
                    pltpu.sync_copy(acc.at[pl.ds(row0, ZR)],
                                    out_hbm.at[c].at[pl.ds(row0, ZR)])

                @pl.when(sid == NSUB - 1)
                def _():
                    pltpu.sync_copy(
                        acc.at[pl.ds((NSUB - 1) * ZR, _TAILR)],
                        out_hbm.at[c].at[pl.ds((NSUB - 1) * ZR, _TAILR)])

                plsc.subcore_barrier()

    return scatter_k


def _make_sc_count():
    @functools.partial(
        pl.kernel,
        out_type=jax.ShapeDtypeStruct((2, N, 128), jnp.float32),
        mesh=_sc_mesh(),
        scratch_types=[
            pltpu.VMEM((EB,), jnp.int32),
            pltpu.VMEM((EB, 128), jnp.float32),
            pltpu.VMEM_SHARED((NPAD, 128), jnp.float32),
        ],
    )
    def count_k(dstc_hbm, srcc_hbm, ones_hbm, zeros_hbm, out_hbm,
                idxv, onesv, acc):
        cid = lax.axis_index("c")
        sid = lax.axis_index("s")
        pltpu.sync_copy(ones_hbm, onesv)
        base = sid * EPT
        row0 = sid * ZR
        pltpu.sync_copy(zeros_hbm, acc.at[pl.ds(row0, ZR)])
        plsc.subcore_barrier()
        for which in range(2):
            @pl.when(cid == which)
            def _():
                ihbm = dstc_hbm if which == 0 else srcc_hbm

                def body(b, carry):
                    off = base + b * EB
                    pltpu.sync_copy(ihbm.at[pl.ds(off, EB)], idxv)
                    pltpu.sync_copy(onesv, acc.at[idxv], add=True)
                    return carry

                lax.fori_loop(0, NBATCH, body, 0)
                plsc.subcore_barrier()

                @pl.when(sid < NSUB - 1)
                def _():
                    pltpu.sync_copy(acc.at[pl.ds(row0, ZR)],
                                    out_hbm.at[which].at[pl.ds(row0, ZR)])

                @pl.when(sid == NSUB - 1)
                def _():
                    pltpu.sync_copy(
                        acc.at[pl.ds((NSUB - 1) * ZR, _TAILR)],
                        out_hbm.at[which].at[pl.ds((NSUB - 1) * ZR, _TAILR)])

    return count_k


_sc_scatter_cached = functools.lru_cache(maxsize=None)(_make_sc_scatter)
_sc_count_cached = functools.lru_cache(maxsize=None)(_make_sc_count)


def _mm(a, b):
    return jax.lax.dot_general(a, b, (((1,), (0,)), ((), ())),
                               preferred_element_type=jnp.float32)


def _ka_body(x_ref, w1_ref, sout_ref, u_ref, yhat_ref):
    u = _mm(x_ref[...], w1_ref[...])
    u_ref[...] = u
    us = u * sout_ref[...]
    for k in range(C1):
        yhat_ref[k] = us[:, k * 128:(k + 1) * 128]


def _kernel_a(x, w1, s_out):
    return pl.pallas_call(
        _ka_body,
        grid=(G,),
        in_specs=[
            pl.BlockSpec((R, D), lambda i: (i, 0)),
            pl.BlockSpec((D, H), lambda i: (0, 0)),
            pl.BlockSpec((R, 1), lambda i: (i, 0)),
        ],
        out_specs=[
            pl.BlockSpec((R, H), lambda i: (i, 0)),
            pl.BlockSpec((C1, R, 128), lambda i: (0, i, 0)),
        ],
        out_shape=[
            jax.ShapeDtypeStruct((N, H), jnp.float32),
            jax.ShapeDtypeStruct((C1, N, 128), jnp.float32),
        ],
    )(x, w1, s_out)


def _kb_body(t1_ref, u_ref, rin_ref, d_ref, sout_ref, w2_ref,
             glp_ref, ghp_ref, yhat_ref):
    u = u_ref[...]
    s1 = jnp.concatenate([t1_ref[k] for k in range(C1)], axis=1) * rin_ref[...]
    h_lp = jnp.maximum(s1 + d_ref[...] * u, 0.0)
    h_hp = jnp.maximum(u - s1, 0.0)
    w2 = w2_ref[...]
    g_lp = _mm(h_lp, w2)
    g_hp = _mm(h_hp, w2)
    glp_ref[...] = g_lp
    ghp_ref[...] = g_hp
    sout = sout_ref[...]
    gs_lp = g_lp * sout
    gs_hp = g_hp * sout
    for k in range(C1):
        yhat_ref[k] = gs_lp[:, k * 128:(k + 1) * 128]
        yhat_ref[C1 + k] = gs_hp[:, k * 128:(k + 1) * 128]


def _kernel_b(t1, u, r_in, d, s_out, w2):
    return pl.pallas_call(
        _kb_body,
        grid=(G,),
        in_specs=[
            pl.BlockSpec((C1, R, 128), lambda i: (0, i, 0)),
            pl.BlockSpec((R, H), lambda i: (i, 0)),
            pl.BlockSpec((R, 1), lambda i: (i, 0)),
            pl.BlockSpec((R, 1), lambda i: (i, 0)),
            pl.BlockSpec((R, 1), lambda i: (i, 0)),
            pl.BlockSpec((H, H), lambda i: (0, 0)),
        ],
        out_specs=[
            pl.BlockSpec((R, H), lambda i: (i, 0)),
            pl.BlockSpec((R, H), lambda i: (i, 0)),
            pl.BlockSpec((C2, R, 128), lambda i: (0, i, 0)),
        ],
        out_shape=[
            jax.ShapeDtypeStruct((N, H), jnp.float32),
            jax.ShapeDtypeStruct((N, H), jnp.float32),
            jax.ShapeDtypeStruct((C2, N, 128), jnp.float32),
        ],
    )(t1, u, r_in, d, s_out, w2)


def _kc_body(t2_ref, glp_ref, ghp_ref, rin_ref, d_ref,
             p1w_ref, p1b_ref, p2w_ref, p2b_ref,
             zlp_ref, zhp_ref, y_ref, sum_ref, sq_ref):
    rin = rin_ref[...]
    z_lp = (jnp.concatenate([t2_ref[k] for k in range(C1)], axis=1) * rin
            + d_ref[...] * glp_ref[...])
    z_hp = (ghp_ref[...]
            - jnp.concatenate([t2_ref[C1 + k] for k in range(C1)], axis=1) * rin)
    zlp_ref[...] = z_lp
    zhp_ref[...] = z_hp
    p1w, p1b = p1w_ref[...], p1b_ref[...]
    p2w, p2b = p2w_ref[...], p2b_ref[...]
    ys = (_mm(z_lp, p1w) + p1b, _mm(z_hp, p1w) + p1b,
          _mm(z_lp, p2w) + p2b, _mm(z_hp, p2w) + p2b)
    for j in range(4):
        y_ref[j] = ys[j]
        sum_ref[0, j] = jnp.sum(ys[j], axis=0, keepdims=True)
        sq_ref[0, j] = jnp.sum(ys[j] * ys[j], axis=0, keepdims=True)


def _kernel_c(t2, g_lp, g_hp, r_in, d, p1w, p1b, p2w, p2b):
    return pl.pallas_call(
        _kc_body,
        grid=(G,),
        in_specs=[
            pl.BlockSpec((C2, R, 128), lambda i: (0, i, 0)),
            pl.BlockSpec((R, H), lambda i: (i, 0)),
            pl.BlockSpec((R, H), lambda i: (i, 0)),
            pl.BlockSpec((R, 1), lambda i: (i, 0)),
            pl.BlockSpec((R, 1), lambda i: (i, 0)),
            pl.BlockSpec((H, H), lambda i: (0, 0)),
            pl.BlockSpec((1, H), lambda i: (0, 0)),
            pl.BlockSpec((H, H), lambda i: (0, 0)),
            pl.BlockSpec((1, H), lambda i: (0, 0)),
        ],
        out_specs=[
            pl.BlockSpec((R, H), lambda i: (i, 0)),
            pl.BlockSpec((R, H), lambda i: (i, 0)),
            pl.BlockSpec((4, R, H), lambda i: (0, i, 0)),
            pl.BlockSpec((1, 4, 1, H), lambda i: (i, 0, 0, 0)),
            pl.BlockSpec((1, 4, 1, H), lambda i: (i, 0, 0, 0)),
        ],
        out_shape=[
            jax.ShapeDtypeStruct((N, H), jnp.float32),
            jax.ShapeDtypeStruct((N, H), jnp.float32),
            jax.ShapeDtypeStruct((4, N, H), jnp.float32),
            jax.ShapeDtypeStruct((G, 4, 1, H), jnp.float32),
            jax.ShapeDtypeStruct((G, 4, 1, H), jnp.float32),
        ],
    )(t2, g_lp, g_hp, r_in, d, p1w, p1b, p2w, p2b)


def _kd_body(y_ref, sum_ref, sq_ref, gb_ref, a_ref, out_ref):
    inv_n = jnp.float32(1.0 / N)
    for j in range(4):
        mu = jnp.sum(sum_ref[:, j], axis=0, keepdims=False) * inv_n
        var = jnp.sum(sq_ref[:, j], axis=0, keepdims=False) * inv_n - mu * mu
        gamma = gb_ref[2 * j][None]
        beta = gb_ref[2 * j + 1][None]
        a = a_ref[j, 0]
        yn = (y_ref[j] - mu) * jax.lax.rsqrt(var + 1e-5) * gamma + beta
        out_ref[j] = jnp.where(yn > 0, yn, a * yn)


def _kernel_d(y, sums, sqs, gb, avec):
    return pl.pallas_call(
        _kd_body,
        grid=(G,),
        in_specs=[
            pl.BlockSpec((4, R, H), lambda i: (0, i, 0)),
            pl.BlockSpec((G, 4, 1, H), lambda i: (0, 0, 0, 0)),
            pl.BlockSpec((G, 4, 1, H), lambda i: (0, 0, 0, 0)),
            pl.BlockSpec((8, H), lambda i: (0, 0)),
            pl.BlockSpec((4, 1), lambda i: (0, 0)),
        ],
        out_specs=pl.BlockSpec((4, R, H), lambda i: (0, i, 0)),
        out_shape=jax.ShapeDtypeStruct((4, N, H), jnp.float32),
    )(y, sums, sqs, gb, avec)


def kernel(x, edge_index, W1, W2, p1_W, p1_b, p1_gamma, p1_beta, p1_a,
           p2_W, p2_b, p2_gamma, p2_beta, p2_a):
    src = edge_index[0].astype(jnp.int32)
    dst = edge_index[1].astype(jnp.int32)

    npadE = EPAD - E
    src_pad = jnp.concatenate([src, jnp.zeros((npadE,), jnp.int32)])
    dst_pad = jnp.concatenate([dst, jnp.full((npadE,), N, jnp.int32)])
    src_cnt = jnp.concatenate([src, jnp.full((npadE,), N, jnp.int32)])

    ones128 = jnp.ones((EB, 128), jnp.float32)
    zeros128 = jnp.zeros((ZR, 128), jnp.float32)

    cnts = _sc_count_cached()(dst_pad, src_cnt, ones128, zeros128)
    cnt_in = cnts[0, :, 0] + 1.0
    cnt_out = cnts[1, :, 0] + 1.0

    r_out = jax.lax.rsqrt(cnt_out)[:, None]
    r_in = jax.lax.rsqrt(cnt_in)[:, None]
    d = r_out * r_in

    u, yhat1 = _kernel_a(x, W1, r_out)
    t1 = _sc_scatter_cached(C1)(src_pad, dst_pad, zeros128, yhat1)
    g_lp, g_hp, yhat2 = _kernel_b(t1, u, r_in, d, r_out, W2)
    t2 = _sc_scatter_cached(C2)(src_pad, dst_pad, zeros128, yhat2)
    z_lp, z_hp, y, sums, sqs = _kernel_c(
        t2, g_lp, g_hp, r_in, d, p1_W, p1_b[None], p2_W, p2_b[None])

    gb = jnp.stack([p1_gamma, p1_beta, p1_gamma, p1_beta,
                    p2_gamma, p2_beta, p2_gamma, p2_beta]).reshape(8, H)
    avec = jnp.stack([p1_a, p1_a, p2_a, p2_a]).reshape(4, 1)
    out = _kernel_d(y, sums, sqs, gb, avec)

    return (out[0], out[1], out[2], out[3], z_lp, z_hp)


# speedup vs baseline: 14.0962x; 1.0048x over previous
import functools

import jax
import jax.numpy as jnp
from jax import lax
from jax.experimental import pallas as pl
from jax.experimental.pallas import tpu as pltpu
from jax.experimental.pallas import tpu_sc as plsc

N, D, H, E = 10000, 256, 512, 160000
R = 1000
G = N // R
C1 = H // 128
C2 = 2 * C1

NSUB = 16
EB = 128
EPAD = 32 * EB * 40
EPT = EPAD // NSUB
NBATCH = EPT // EB
ZR = 632
NPAD = NSUB * ZR
_TAILR = N - (NSUB - 1) * ZR


def _sc_mesh():
    return plsc.VectorSubcoreMesh(core_axis_name="c", subcore_axis_name="s",
                                  num_cores=2, num_subcores=NSUB)


def _make_sc_scatter(K):
    Kh = K // 2

    @functools.partial(
        pl.kernel,
        out_type=jax.ShapeDtypeStruct((K, N, 128), jnp.float32),
        mesh=_sc_mesh(),
        scratch_types=[
            pltpu.VMEM((EB,), jnp.int32),
            pltpu.VMEM((EB,), jnp.int32),
            pltpu.VMEM((EB,), jnp.int32),
            pltpu.VMEM((EB,), jnp.int32),
            pltpu.VMEM((EB, 128), jnp.float32),
            pltpu.VMEM((EB, 128), jnp.float32),
            pltpu.VMEM_SHARED((NPAD, 128), jnp.float32),
            pltpu.SemaphoreType.DMA,
            pltpu.SemaphoreType.DMA,
            pltpu.SemaphoreType.DMA,
        ],
    )
    def scatter_k(src_hbm, dst_hbm, zeros_hbm, y_hbm, out_hbm,
                  sidx0, sidx1, didx0, didx1, rows0, rows1, acc, sem,
                  gsem0, gsem1):
        cid = lax.axis_index("c")
        sid = lax.axis_index("s")
        row0 = sid * ZR
        base = sid * EPT
        sbuf = (sidx0, sidx1)
        dbuf = (didx0, didx1)
        rbuf = (rows0, rows1)

        for c in range(K):
            @pl.when(cid == (c // Kh))
            def _():
                pltpu.sync_copy(zeros_hbm, acc.at[pl.ds(row0, ZR)])
                plsc.subcore_barrier()
                yc = y_hbm.at[c]

                def body2(i, carry):
                    o0 = base + (2 * i) * EB
                    o1 = o0 + EB
                    i0 = pltpu.async_copy(src_hbm.at[pl.ds(o0, EB)],
                                          sbuf[0], sem)
                    i1 = pltpu.async_copy(dst_hbm.at[pl.ds(o0, EB)],
                                          dbuf[0], sem)
                    i2 = pltpu.async_copy(src_hbm.at[pl.ds(o1, EB)],
                                          sbuf[1], sem)
                    i3 = pltpu.async_copy(dst_hbm.at[pl.ds(o1, EB)],
                                          dbuf[1], sem)
                    i0.wait()
                    i1.wait()
                    i2.wait()
                    i3.wait()
                    g0 = pltpu.async_copy(yc.at[sbuf[0]], rbuf[0], gsem0)
                    g1 = pltpu.async_copy(yc.at[sbuf[1]], rbuf[1], gsem1)
                    g0.wait()
                    pltpu.sync_copy(rbuf[0], acc.at[dbuf[0]], add=True)
                    g1.wait()
                    pltpu.sync_copy(rbuf[1], acc.at[dbuf[1]], add=True)
                    return carry

                lax.fori_loop(0, NBATCH // 2, body2, 0)
                plsc.subcore_barrier()

                @pl.when(sid < NSUB - 1)
                def _():
                    pltpu.sync_copy(acc.at[pl.ds(row0, ZR)],
                                    out_hbm.at[c].at[pl.ds(row0, ZR)])

                @pl.when(sid == NSUB - 1)
                def _():
                    pltpu.sync_copy(
                        acc.at[pl.ds((NSUB - 1) * ZR, _TAILR)],
                        out_hbm.at[c].at[pl.ds((NSUB - 1) * ZR, _TAILR)])

                plsc.subcore_barrier()

    return scatter_k


def _make_sc_count():
    @functools.partial(
        pl.kernel,
        out_type=jax.ShapeDtypeStruct((2, N, 128), jnp.float32),
        mesh=_sc_mesh(),
        scratch_types=[
            pltpu.VMEM((EB,), jnp.int32),
            pltpu.VMEM((EB, 128), jnp.float32),
            pltpu.VMEM_SHARED((NPAD, 128), jnp.float32),
        ],
    )
    def count_k(dstc_hbm, srcc_hbm, ones_hbm, zeros_hbm, out_hbm,
                idxv, onesv, acc):
        cid = lax.axis_index("c")
        sid = lax.axis_index("s")
        pltpu.sync_copy(ones_hbm, onesv)
        base = sid * EPT
        row0 = sid * ZR
        pltpu.sync_copy(zeros_hbm, acc.at[pl.ds(row0, ZR)])
        plsc.subcore_barrier()
        for which in range(2):
            @pl.when(cid == which)
            def _():
                ihbm = dstc_hbm if which == 0 else srcc_hbm

                def body(b, carry):
                    off = base + b * EB
                    pltpu.sync_copy(ihbm.at[pl.ds(off, EB)], idxv)
                    pltpu.sync_copy(onesv, acc.at[idxv], add=True)
                    return carry

                lax.fori_loop(0, NBATCH, body, 0)
                plsc.subcore_barrier()

                @pl.when(sid < NSUB - 1)
                def _():
                    pltpu.sync_copy(acc.at[pl.ds(row0, ZR)],
                                    out_hbm.at[which].at[pl.ds(row0, ZR)])

                @pl.when(sid == NSUB - 1)
                def _():
                    pltpu.sync_copy(
                        acc.at[pl.ds((NSUB - 1) * ZR, _TAILR)],
                        out_hbm.at[which].at[pl.ds((NSUB - 1) * ZR, _TAILR)])

    return count_k


_sc_scatter_cached = functools.lru_cache(maxsize=None)(_make_sc_scatter)
_sc_count_cached = functools.lru_cache(maxsize=None)(_make_sc_count)


def _mm(a, b):
    return jax.lax.dot_general(a, b, (((1,), (0,)), ((), ())),
                               preferred_element_type=jnp.float32)


def _ka_body(x_ref, w1_ref, sout_ref, u_ref, yhat_ref):
    u = _mm(x_ref[...], w1_ref[...])
    u_ref[...] = u
    us = u * sout_ref[...]
    for k in range(C1):
        yhat_ref[k] = us[:, k * 128:(k + 1) * 128]


def _kernel_a(x, w1, s_out):
    return pl.pallas_call(
        _ka_body,
        grid=(G,),
        in_specs=[
            pl.BlockSpec((R, D), lambda i: (i, 0)),
            pl.BlockSpec((D, H), lambda i: (0, 0)),
            pl.BlockSpec((R, 1), lambda i: (i, 0)),
        ],
        out_specs=[
            pl.BlockSpec((R, H), lambda i: (i, 0)),
            pl.BlockSpec((C1, R, 128), lambda i: (0, i, 0)),
        ],
        out_shape=[
            jax.ShapeDtypeStruct((N, H), jnp.float32),
            jax.ShapeDtypeStruct((C1, N, 128), jnp.float32),
        ],
    )(x, w1, s_out)


def _kb_body(t1_ref, u_ref, rin_ref, d_ref, sout_ref, w2_ref,
             glp_ref, ghp_ref, yhat_ref):
    u = u_ref[...]
    s1 = jnp.concatenate([t1_ref[k] for k in range(C1)], axis=1) * rin_ref[...]
    h_lp = jnp.maximum(s1 + d_ref[...] * u, 0.0)
    h_hp = jnp.maximum(u - s1, 0.0)
    w2 = w2_ref[...]
    g_lp = _mm(h_lp, w2)
    g_hp = _mm(h_hp, w2)
    glp_ref[...] = g_lp
    ghp_ref[...] = g_hp
    sout = sout_ref[...]
    gs_lp = g_lp * sout
    gs_hp = g_hp * sout
    for k in range(C1):
        yhat_ref[k] = gs_lp[:, k * 128:(k + 1) * 128]
        yhat_ref[C1 + k] = gs_hp[:, k * 128:(k + 1) * 128]


def _kernel_b(t1, u, r_in, d, s_out, w2):
    return pl.pallas_call(
        _kb_body,
        grid=(G,),
        in_specs=[
            pl.BlockSpec((C1, R, 128), lambda i: (0, i, 0)),
            pl.BlockSpec((R, H), lambda i: (i, 0)),
            pl.BlockSpec((R, 1), lambda i: (i, 0)),
            pl.BlockSpec((R, 1), lambda i: (i, 0)),
            pl.BlockSpec((R, 1), lambda i: (i, 0)),
            pl.BlockSpec((H, H), lambda i: (0, 0)),
        ],
        out_specs=[
            pl.BlockSpec((R, H), lambda i: (i, 0)),
            pl.BlockSpec((R, H), lambda i: (i, 0)),
            pl.BlockSpec((C2, R, 128), lambda i: (0, i, 0)),
        ],
        out_shape=[
            jax.ShapeDtypeStruct((N, H), jnp.float32),
            jax.ShapeDtypeStruct((N, H), jnp.float32),
            jax.ShapeDtypeStruct((C2, N, 128), jnp.float32),
        ],
    )(t1, u, r_in, d, s_out, w2)


def _kc_body(t2_ref, glp_ref, ghp_ref, rin_ref, d_ref,
             p1w_ref, p1b_ref, p2w_ref, p2b_ref,
             zlp_ref, zhp_ref, y_ref, sum_ref, sq_ref):
    rin = rin_ref[...]
    z_lp = (jnp.concatenate([t2_ref[k] for k in range(C1)], axis=1) * rin
            + d_ref[...] * glp_ref[...])
    z_hp = (ghp_ref[...]
            - jnp.concatenate([t2_ref[C1 + k] for k in range(C1)], axis=1) * rin)
    zlp_ref[...] = z_lp
    zhp_ref[...] = z_hp
    p1w, p1b = p1w_ref[...], p1b_ref[...]
    p2w, p2b = p2w_ref[...], p2b_ref[...]
    ys = (_mm(z_lp, p1w) + p1b, _mm(z_hp, p1w) + p1b,
          _mm(z_lp, p2w) + p2b, _mm(z_hp, p2w) + p2b)
    for j in range(4):
        y_ref[j] = ys[j]
        sum_ref[0, j] = jnp.sum(ys[j], axis=0, keepdims=True)
        sq_ref[0, j] = jnp.sum(ys[j] * ys[j], axis=0, keepdims=True)


def _kernel_c(t2, g_lp, g_hp, r_in, d, p1w, p1b, p2w, p2b):
    return pl.pallas_call(
        _kc_body,
        grid=(G,),
        in_specs=[
            pl.BlockSpec((C2, R, 128), lambda i: (0, i, 0)),
            pl.BlockSpec((R, H), lambda i: (i, 0)),
            pl.BlockSpec((R, H), lambda i: (i, 0)),
            pl.BlockSpec((R, 1), lambda i: (i, 0)),
            pl.BlockSpec((R, 1), lambda i: (i, 0)),
            pl.BlockSpec((H, H), lambda i: (0, 0)),
            pl.BlockSpec((1, H), lambda i: (0, 0)),
            pl.BlockSpec((H, H), lambda i: (0, 0)),
            pl.BlockSpec((1, H), lambda i: (0, 0)),
        ],
        out_specs=[
            pl.BlockSpec((R, H), lambda i: (i, 0)),
            pl.BlockSpec((R, H), lambda i: (i, 0)),
            pl.BlockSpec((4, R, H), lambda i: (0, i, 0)),
            pl.BlockSpec((1, 4, 1, H), lambda i: (i, 0, 0, 0)),
            pl.BlockSpec((1, 4, 1, H), lambda i: (i, 0, 0, 0)),
        ],
        out_shape=[
            jax.ShapeDtypeStruct((N, H), jnp.float32),
            jax.ShapeDtypeStruct((N, H), jnp.float32),
            jax.ShapeDtypeStruct((4, N, H), jnp.float32),
            jax.ShapeDtypeStruct((G, 4, 1, H), jnp.float32),
            jax.ShapeDtypeStruct((G, 4, 1, H), jnp.float32),
        ],
    )(t2, g_lp, g_hp, r_in, d, p1w, p1b, p2w, p2b)


def _kd_body(y_ref, sum_ref, sq_ref, gb_ref, a_ref, out_ref):
    inv_n = jnp.float32(1.0 / N)
    for j in range(4):
        mu = jnp.sum(sum_ref[:, j], axis=0, keepdims=False) * inv_n
        var = jnp.sum(sq_ref[:, j], axis=0, keepdims=False) * inv_n - mu * mu
        gamma = gb_ref[2 * j][None]
        beta = gb_ref[2 * j + 1][None]
        a = a_ref[j, 0]
        yn = (y_ref[j] - mu) * jax.lax.rsqrt(var + 1e-5) * gamma + beta
        out_ref[j] = jnp.where(yn > 0, yn, a * yn)


def _kernel_d(y, sums, sqs, gb, avec):
    return pl.pallas_call(
        _kd_body,
        grid=(G,),
        in_specs=[
            pl.BlockSpec((4, R, H), lambda i: (0, i, 0)),
            pl.BlockSpec((G, 4, 1, H), lambda i: (0, 0, 0, 0)),
            pl.BlockSpec((G, 4, 1, H), lambda i: (0, 0, 0, 0)),
            pl.BlockSpec((8, H), lambda i: (0, 0)),
            pl.BlockSpec((4, 1), lambda i: (0, 0)),
        ],
        out_specs=pl.BlockSpec((4, R, H), lambda i: (0, i, 0)),
        out_shape=jax.ShapeDtypeStruct((4, N, H), jnp.float32),
    )(y, sums, sqs, gb, avec)


def kernel(x, edge_index, W1, W2, p1_W, p1_b, p1_gamma, p1_beta, p1_a,
           p2_W, p2_b, p2_gamma, p2_beta, p2_a):
    src = edge_index[0].astype(jnp.int32)
    dst = edge_index[1].astype(jnp.int32)

    npadE = EPAD - E
    src_pad = jnp.concatenate([src, jnp.zeros((npadE,), jnp.int32)])
    dst_pad = jnp.concatenate([dst, jnp.full((npadE,), N, jnp.int32)])
    src_cnt = jnp.concatenate([src, jnp.full((npadE,), N, jnp.int32)])

    ones128 = jnp.ones((EB, 128), jnp.float32)
    zeros128 = jnp.zeros((ZR, 128), jnp.float32)

    cnts = _sc_count_cached()(dst_pad, src_cnt, ones128, zeros128)
    cnt_in = cnts[0, :, 0] + 1.0
    cnt_out = cnts[1, :, 0] + 1.0

    r_out = jax.lax.rsqrt(cnt_out)[:, None]
    r_in = jax.lax.rsqrt(cnt_in)[:, None]
    d = r_out * r_in

    u, yhat1 = _kernel_a(x, W1, r_out)
    t1 = _sc_scatter_cached(C1)(src_pad, dst_pad, zeros128, yhat1)
    g_lp, g_hp, yhat2 = _kernel_b(t1, u, r_in, d, r_out, W2)
    t2 = _sc_scatter_cached(C2)(src_pad, dst_pad, zeros128, yhat2)
    z_lp, z_hp, y, sums, sqs = _kernel_c(
        t2, g_lp, g_hp, r_in, d, p1_W, p1_b[None], p2_W, p2_b[None])

    gb = jnp.stack([p1_gamma, p1_beta, p1_gamma, p1_beta,
                    p2_gamma, p2_beta, p2_gamma, p2_beta]).reshape(8, H)
    avec = jnp.stack([p1_a, p1_a, p2_a, p2_a]).reshape(4, 1)
    out = _kernel_d(y, sums, sqs, gb, avec)

    return (out[0], out[1], out[2], out[3], z_lp, z_hp)


# speedup vs baseline: 14.1732x; 1.0055x over previous
import functools

import jax
import jax.numpy as jnp
from jax import lax
from jax.experimental import pallas as pl
from jax.experimental.pallas import tpu as pltpu
from jax.experimental.pallas import tpu_sc as plsc

N, D, H, E = 10000, 256, 512, 160000
R = 1000
G = N // R
C1 = H // 128
C2 = 2 * C1

NSUB = 16
EB = 128
EPAD = 32 * EB * 40
EPT = EPAD // NSUB
NBATCH = EPT // EB
ZR = 632
NPAD = NSUB * ZR
_TAILR = N - (NSUB - 1) * ZR


def _sc_mesh():
    return plsc.VectorSubcoreMesh(core_axis_name="c", subcore_axis_name="s",
                                  num_cores=2, num_subcores=NSUB)


def _make_sc_scatter(K):
    Kh = K // 2

    @functools.partial(
        pl.kernel,
        out_type=jax.ShapeDtypeStruct((K, N, 128), jnp.float32),
        mesh=_sc_mesh(),
        scratch_types=[
            pltpu.VMEM((EB,), jnp.int32),
            pltpu.VMEM((EB,), jnp.int32),
            pltpu.VMEM((EB,), jnp.int32),
            pltpu.VMEM((EB,), jnp.int32),
            pltpu.VMEM((EB, 128), jnp.float32),
            pltpu.VMEM((EB, 128), jnp.float32),
            pltpu.VMEM_SHARED((NPAD, 128), jnp.float32),
            pltpu.SemaphoreType.DMA,
            pltpu.SemaphoreType.DMA,
            pltpu.SemaphoreType.DMA,
            pltpu.SemaphoreType.DMA,
            pltpu.SemaphoreType.DMA,
        ],
    )
    def scatter_k(src_hbm, dst_hbm, zeros_hbm, y_hbm, out_hbm,
                  sidx0, sidx1, didx0, didx1, rows0, rows1, acc, sem,
                  gsem0, gsem1, ssem0, ssem1):
        cid = lax.axis_index("c")
        sid = lax.axis_index("s")
        row0 = sid * ZR
        base = sid * EPT
        sbuf = (sidx0, sidx1)
        dbuf = (didx0, didx1)
        rbuf = (rows0, rows1)

        for c in range(K):
            @pl.when(cid == (c // Kh))
            def _():
                pltpu.sync_copy(zeros_hbm, acc.at[pl.ds(row0, ZR)])
                plsc.subcore_barrier()
                yc = y_hbm.at[c]

                def body2(i, carry):
                    o0 = base + (2 * i) * EB
                    o1 = o0 + EB
                    i0 = pltpu.async_copy(src_hbm.at[pl.ds(o0, EB)],
                                          sbuf[0], sem)
                    i1 = pltpu.async_copy(dst_hbm.at[pl.ds(o0, EB)],
                                          dbuf[0], sem)
                    i2 = pltpu.async_copy(src_hbm.at[pl.ds(o1, EB)],
                                          sbuf[1], sem)
                    i3 = pltpu.async_copy(dst_hbm.at[pl.ds(o1, EB)],
                                          dbuf[1], sem)
                    i0.wait()
                    i1.wait()
                    i2.wait()
                    i3.wait()
                    g0 = pltpu.async_copy(yc.at[sbuf[0]], rbuf[0], gsem0)
                    g1 = pltpu.async_copy(yc.at[sbuf[1]], rbuf[1], gsem1)
                    g0.wait()
                    s0 = pltpu.async_copy(rbuf[0], acc.at[dbuf[0]], ssem0,
                                          add=True)
                    g1.wait()
                    s1 = pltpu.async_copy(rbuf[1], acc.at[dbuf[1]], ssem1,
                                          add=True)
                    s0.wait()
                    s1.wait()
                    return carry

                lax.fori_loop(0, NBATCH // 2, body2, 0)
                plsc.subcore_barrier()

                @pl.when(sid < NSUB - 1)
                def _():
                    pltpu.sync_copy(acc.at[pl.ds(row0, ZR)],
                                    out_hbm.at[c].at[pl.ds(row0, ZR)])

                @pl.when(sid == NSUB - 1)
                def _():
                    pltpu.sync_copy(
                        acc.at[pl.ds((NSUB - 1) * ZR, _TAILR)],
                        out_hbm.at[c].at[pl.ds((NSUB - 1) * ZR, _TAILR)])

                plsc.subcore_barrier()

    return scatter_k


def _make_sc_count():
    @functools.partial(
        pl.kernel,
        out_type=jax.ShapeDtypeStruct((2, N, 128), jnp.float32),
        mesh=_sc_mesh(),
        scratch_types=[
            pltpu.VMEM((EB,), jnp.int32),
            pltpu.VMEM((EB, 128), jnp.float32),
            pltpu.VMEM_SHARED((NPAD, 128), jnp.float32),
        ],
    )
    def count_k(dstc_hbm, srcc_hbm, ones_hbm, zeros_hbm, out_hbm,
                idxv, onesv, acc):
        cid = lax.axis_index("c")
        sid = lax.axis_index("s")
        pltpu.sync_copy(ones_hbm, onesv)
        base = sid * EPT
        row0 = sid * ZR
        pltpu.sync_copy(zeros_hbm, acc.at[pl.ds(row0, ZR)])
        plsc.subcore_barrier()
        for which in range(2):
            @pl.when(cid == which)
            def _():
                ihbm = dstc_hbm if which == 0 else srcc_hbm

                def body(b, carry):
                    off = base + b * EB
                    pltpu.sync_copy(ihbm.at[pl.ds(off, EB)], idxv)
                    pltpu.sync_copy(onesv, acc.at[idxv], add=True)
                    return carry

                lax.fori_loop(0, NBATCH, body, 0)
                plsc.subcore_barrier()

                @pl.when(sid < NSUB - 1)
                def _():
                    pltpu.sync_copy(acc.at[pl.ds(row0, ZR)],
                                    out_hbm.at[which].at[pl.ds(row0, ZR)])

                @pl.when(sid == NSUB - 1)
                def _():
                    pltpu.sync_copy(
                        acc.at[pl.ds((NSUB - 1) * ZR, _TAILR)],
                        out_hbm.at[which].at[pl.ds((NSUB - 1) * ZR, _TAILR)])

    return count_k


_sc_scatter_cached = functools.lru_cache(maxsize=None)(_make_sc_scatter)
_sc_count_cached = functools.lru_cache(maxsize=None)(_make_sc_count)


def _mm(a, b):
    return jax.lax.dot_general(a, b, (((1,), (0,)), ((), ())),
                               preferred_element_type=jnp.float32)


def _ka_body(x_ref, w1_ref, sout_ref, u_ref, yhat_ref):
    u = _mm(x_ref[...], w1_ref[...])
    u_ref[...] = u
    us = u * sout_ref[...]
    for k in range(C1):
        yhat_ref[k] = us[:, k * 128:(k + 1) * 128]


def _kernel_a(x, w1, s_out):
    return pl.pallas_call(
        _ka_body,
        grid=(G,),
        in_specs=[
            pl.BlockSpec((R, D), lambda i: (i, 0)),
            pl.BlockSpec((D, H), lambda i: (0, 0)),
            pl.BlockSpec((R, 1), lambda i: (i, 0)),
        ],
        out_specs=[
            pl.BlockSpec((R, H), lambda i: (i, 0)),
            pl.BlockSpec((C1, R, 128), lambda i: (0, i, 0)),
        ],
        out_shape=[
            jax.ShapeDtypeStruct((N, H), jnp.float32),
            jax.ShapeDtypeStruct((C1, N, 128), jnp.float32),
        ],
    )(x, w1, s_out)


def _kb_body(t1_ref, u_ref, rin_ref, d_ref, sout_ref, w2_ref,
             glp_ref, ghp_ref, yhat_ref):
    u = u_ref[...]
    s1 = jnp.concatenate([t1_ref[k] for k in range(C1)], axis=1) * rin_ref[...]
    h_lp = jnp.maximum(s1 + d_ref[...] * u, 0.0)
    h_hp = jnp.maximum(u - s1, 0.0)
    w2 = w2_ref[...]
    g_lp = _mm(h_lp, w2)
    g_hp = _mm(h_hp, w2)
    glp_ref[...] = g_lp
    ghp_ref[...] = g_hp
    sout = sout_ref[...]
    gs_lp = g_lp * sout
    gs_hp = g_hp * sout
    for k in range(C1):
        yhat_ref[k] = gs_lp[:, k * 128:(k + 1) * 128]
        yhat_ref[C1 + k] = gs_hp[:, k * 128:(k + 1) * 128]


def _kernel_b(t1, u, r_in, d, s_out, w2):
    return pl.pallas_call(
        _kb_body,
        grid=(G,),
        in_specs=[
            pl.BlockSpec((C1, R, 128), lambda i: (0, i, 0)),
            pl.BlockSpec((R, H), lambda i: (i, 0)),
            pl.BlockSpec((R, 1), lambda i: (i, 0)),
            pl.BlockSpec((R, 1), lambda i: (i, 0)),
            pl.BlockSpec((R, 1), lambda i: (i, 0)),
            pl.BlockSpec((H, H), lambda i: (0, 0)),
        ],
        out_specs=[
            pl.BlockSpec((R, H), lambda i: (i, 0)),
            pl.BlockSpec((R, H), lambda i: (i, 0)),
            pl.BlockSpec((C2, R, 128), lambda i: (0, i, 0)),
        ],
        out_shape=[
            jax.ShapeDtypeStruct((N, H), jnp.float32),
            jax.ShapeDtypeStruct((N, H), jnp.float32),
            jax.ShapeDtypeStruct((C2, N, 128), jnp.float32),
        ],
    )(t1, u, r_in, d, s_out, w2)


def _kc_body(t2_ref, glp_ref, ghp_ref, rin_ref, d_ref,
             p1w_ref, p1b_ref, p2w_ref, p2b_ref,
             zlp_ref, zhp_ref, y_ref, sum_ref, sq_ref):
    rin = rin_ref[...]
    z_lp = (jnp.concatenate([t2_ref[k] for k in range(C1)], axis=1) * rin
            + d_ref[...] * glp_ref[...])
    z_hp = (ghp_ref[...]
            - jnp.concatenate([t2_ref[C1 + k] for k in range(C1)], axis=1) * rin)
    zlp_ref[...] = z_lp
    zhp_ref[...] = z_hp
    p1w, p1b = p1w_ref[...], p1b_ref[...]
    p2w, p2b = p2w_ref[...], p2b_ref[...]
    ys = (_mm(z_lp, p1w) + p1b, _mm(z_hp, p1w) + p1b,
          _mm(z_lp, p2w) + p2b, _mm(z_hp, p2w) + p2b)
    for j in range(4):
        y_ref[j] = ys[j]
        sum_ref[0, j] = jnp.sum(ys[j], axis=0, keepdims=True)
        sq_ref[0, j] = jnp.sum(ys[j] * ys[j], axis=0, keepdims=True)


def _kernel_c(t2, g_lp, g_hp, r_in, d, p1w, p1b, p2w, p2b):
    return pl.pallas_call(
        _kc_body,
        grid=(G,),
        in_specs=[
            pl.BlockSpec((C2, R, 128), lambda i: (0, i, 0)),
            pl.BlockSpec((R, H), lambda i: (i, 0)),
            pl.BlockSpec((R, H), lambda i: (i, 0)),
            pl.BlockSpec((R, 1), lambda i: (i, 0)),
            pl.BlockSpec((R, 1), lambda i: (i, 0)),
            pl.BlockSpec((H, H), lambda i: (0, 0)),
            pl.BlockSpec((1, H), lambda i: (0, 0)),
            pl.BlockSpec((H, H), lambda i: (0, 0)),
            pl.BlockSpec((1, H), lambda i: (0, 0)),
        ],
        out_specs=[
            pl.BlockSpec((R, H), lambda i: (i, 0)),
            pl.BlockSpec((R, H), lambda i: (i, 0)),
            pl.BlockSpec((4, R, H), lambda i: (0, i, 0)),
            pl.BlockSpec((1, 4, 1, H), lambda i: (i, 0, 0, 0)),
            pl.BlockSpec((1, 4, 1, H), lambda i: (i, 0, 0, 0)),
        ],
        out_shape=[
            jax.ShapeDtypeStruct((N, H), jnp.float32),
            jax.ShapeDtypeStruct((N, H), jnp.float32),
            jax.ShapeDtypeStruct((4, N, H), jnp.float32),
            jax.ShapeDtypeStruct((G, 4, 1, H), jnp.float32),
            jax.ShapeDtypeStruct((G, 4, 1, H), jnp.float32),
        ],
    )(t2, g_lp, g_hp, r_in, d, p1w, p1b, p2w, p2b)


def _kd_body(y_ref, sum_ref, sq_ref, gb_ref, a_ref, out_ref):
    inv_n = jnp.float32(1.0 / N)
    for j in range(4):
        mu = jnp.sum(sum_ref[:, j], axis=0, keepdims=False) * inv_n
        var = jnp.sum(sq_ref[:, j], axis=0, keepdims=False) * inv_n - mu * mu
        gamma = gb_ref[2 * j][None]
        beta = gb_ref[2 * j + 1][None]
        a = a_ref[j, 0]
        yn = (y_ref[j] - mu) * jax.lax.rsqrt(var + 1e-5) * gamma + beta
        out_ref[j] = jnp.where(yn > 0, yn, a * yn)


def _kernel_d(y, sums, sqs, gb, avec):
    return pl.pallas_call(
        _kd_body,
        grid=(G,),
        in_specs=[
            pl.BlockSpec((4, R, H), lambda i: (0, i, 0)),
            pl.BlockSpec((G, 4, 1, H), lambda i: (0, 0, 0, 0)),
            pl.BlockSpec((G, 4, 1, H), lambda i: (0, 0, 0, 0)),
            pl.BlockSpec((8, H), lambda i: (0, 0)),
            pl.BlockSpec((4, 1), lambda i: (0, 0)),
        ],
        out_specs=pl.BlockSpec((4, R, H), lambda i: (0, i, 0)),
        out_shape=jax.ShapeDtypeStruct((4, N, H), jnp.float32),
    )(y, sums, sqs, gb, avec)


def kernel(x, edge_index, W1, W2, p1_W, p1_b, p1_gamma, p1_beta, p1_a,
           p2_W, p2_b, p2_gamma, p2_beta, p2_a):
    src = edge_index[0].astype(jnp.int32)
    dst = edge_index[1].astype(jnp.int32)

    npadE = EPAD - E
    src_pad = jnp.concatenate([src, jnp.zeros((npadE,), jnp.int32)])
    dst_pad = jnp.concatenate([dst, jnp.full((npadE,), N, jnp.int32)])
    src_cnt = jnp.concatenate([src, jnp.full((npadE,), N, jnp.int32)])

    ones128 = jnp.ones((EB, 128), jnp.float32)
    zeros128 = jnp.zeros((ZR, 128), jnp.float32)

    cnts = _sc_count_cached()(dst_pad, src_cnt, ones128, zeros128)
    cnt_in = cnts[0, :, 0] + 1.0
    cnt_out = cnts[1, :, 0] + 1.0

    r_out = jax.lax.rsqrt(cnt_out)[:, None]
    r_in = jax.lax.rsqrt(cnt_in)[:, None]
    d = r_out * r_in

    u, yhat1 = _kernel_a(x, W1, r_out)
    t1 = _sc_scatter_cached(C1)(src_pad, dst_pad, zeros128, yhat1)
    g_lp, g_hp, yhat2 = _kernel_b(t1, u, r_in, d, r_out, W2)
    t2 = _sc_scatter_cached(C2)(src_pad, dst_pad, zeros128, yhat2)
    z_lp, z_hp, y, sums, sqs = _kernel_c(
        t2, g_lp, g_hp, r_in, d, p1_W, p1_b[None], p2_W, p2_b[None])

    gb = jnp.stack([p1_gamma, p1_beta, p1_gamma, p1_beta,
                    p2_gamma, p2_beta, p2_gamma, p2_beta]).reshape(8, H)
    avec = jnp.stack([p1_a, p1_a, p2_a, p2_a]).reshape(4, 1)
    out = _kernel_d(y, sums, sqs, gb, avec)

    return (out[0], out[1], out[2], out[3], z_lp, z_hp)
